# Initial kernel scaffold; baseline (speedup 1.0000x reference)
import jax
import jax.numpy as jnp
from jax.experimental import pallas as pl


def kernel(q_sub, q_rel, hidden, edges, n_node, old_nodes_new_idx, rela_embed, Ws_attn, Wr_attn, Wqr_attn_W, Wqr_attn_b, w_alpha_W, w_alpha_b, W_h):
    raise NotImplementedError("write your pallas kernel here")


# speedup vs baseline: 2.5114x; 2.5114x over previous
import functools

import jax
import jax.numpy as jnp
from jax import lax
from jax.experimental import pallas as pl
from jax.experimental.pallas import tpu as pltpu
from jax.experimental.pallas import tpu_sc as plsc

N = 10000
E = 320000
D = 128
PADW = 16
NC, NS = 2, 16
NTILES = NC * NS
CH = 128
NCHUNKS_ALL = E // CH
RPT = 632
RPT_LAST = N - (NS - 1) * RPT


def _proj_body(hid, re_, ws, wr, wq, bq, ps_o, pr_o, pq_o):
    ps_o[...] = jnp.dot(hid[...], ws[...], preferred_element_type=jnp.float32)
    pr_o[...] = jnp.dot(re_[...], wr[...], preferred_element_type=jnp.float32)
    pq_o[...] = (jnp.dot(re_[...], wq[...], preferred_element_type=jnp.float32)
                 + bq[...])


def _proj_tables(hidden, re_, ws_p, wr_p, wq_p, bq_p):
    bs = 1000
    grid = (N // bs,)
    return pl.pallas_call(
        _proj_body,
        grid=grid,
        in_specs=[
            pl.BlockSpec((bs, D), lambda i: (i, 0)),
            pl.BlockSpec((bs, D), lambda i: (i, 0)),
            pl.BlockSpec((D, PADW), lambda i: (0, 0)),
            pl.BlockSpec((D, PADW), lambda i: (0, 0)),
            pl.BlockSpec((D, PADW), lambda i: (0, 0)),
            pl.BlockSpec((1, PADW), lambda i: (0, 0)),
        ],
        out_specs=[
            pl.BlockSpec((bs, PADW), lambda i: (i, 0)),
            pl.BlockSpec((bs, PADW), lambda i: (i, 0)),
            pl.BlockSpec((bs, PADW), lambda i: (i, 0)),
        ],
        out_shape=[jax.ShapeDtypeStruct((N, PADW), jnp.float32)] * 3,
    )(hidden, re_, ws_p, wr_p, wq_p, bq_p)


def _sc_body(idx_h, qrel_h, hid_h, re_h, wv_h,
             p0_h, p1_h, p2_h, p3_h, p4_h, p5_h, p6_h, p7_h, p8_h, p9_h,
             p10_h, p11_h, p12_h, p13_h, p14_h, out_h,
             idx_v, qi_v, att_v, hs_v, hr_v, alpha_v, wv_v, agg_s):
    planes = (p0_h, p1_h, p2_h, p3_h, p4_h, p5_h, p6_h, p7_h, p8_h, p9_h,
              p10_h, p11_h, p12_h, p13_h, p14_h)
    cid = lax.axis_index("c")
    sid = lax.axis_index("s")
    wid = cid * NS + sid

    zf = jnp.zeros((16,), jnp.float32)

    pltpu.sync_copy(wv_h, wv_v)

    def _zrow(rr, carry):
        for dblk in range(D // 16):
            hs_v[rr, pl.ds(dblk * 16, 16)] = zf
        return carry
    lax.fori_loop(0, CH, _zrow, 0)

    def _zero_rows(start, count):
        nfull = count // CH
        def _zagg(k, carry):
            pltpu.sync_copy(hs_v, agg_s.at[pl.ds(start + k * CH, CH)])
            return carry
        lax.fori_loop(0, nfull, _zagg, 0)
        rem = count - nfull * CH
        if rem:
            pltpu.sync_copy(hs_v.at[pl.ds(0, rem)],
                            agg_s.at[pl.ds(start + nfull * CH, rem)])

    @pl.when(sid < NS - 1)
    def _():
        _zero_rows(sid * RPT, RPT)

    @pl.when(sid == NS - 1)
    def _():
        _zero_rows(sid * RPT, RPT_LAST)
    plsc.subcore_barrier()

    def _chunk(k, carry):
        c = wid + NTILES * k
        pltpu.sync_copy(idx_h.at[c], idx_v)
        sub_s = idx_v.at[0]
        rel_s = idx_v.at[1]
        pltpu.sync_copy(qrel_h.at[idx_v.at[2]], qi_v)
        for j in range(5):
            pltpu.sync_copy(planes[j].at[sub_s], att_v.at[j])
            pltpu.sync_copy(planes[5 + j].at[rel_s], att_v.at[5 + j])
            pltpu.sync_copy(planes[10 + j].at[qi_v], att_v.at[10 + j])
        pltpu.sync_copy(hid_h.at[sub_s], hs_v)
        pltpu.sync_copy(re_h.at[rel_s], hr_v)

        wvec = wv_v[...]
        for g in range(CH // 16):
            logit = jnp.broadcast_to(wvec[5], (16,))
            for j in range(5):
                sj = att_v[j, pl.ds(g * 16, 16)]
                rj = att_v[5 + j, pl.ds(g * 16, 16)]
                qj = att_v[10 + j, pl.ds(g * 16, 16)]
                logit = logit + jnp.maximum(sj + rj + qj, 0.0) * wvec[j]
            alpha = 1.0 / (1.0 + jnp.exp(-logit))
            alpha_v[pl.ds(g * 16, 16)] = alpha

        def _msg(e, c2):
            a = jnp.broadcast_to(alpha_v[pl.ds(e, 16)][0], (16,))
            for dblk in range(D // 16):
                h1 = hs_v[e, pl.ds(dblk * 16, 16)]
                h2 = hr_v[e, pl.ds(dblk * 16, 16)]
                hs_v[e, pl.ds(dblk * 16, 16)] = a * (h1 + h2)
            return c2
        lax.fori_loop(0, CH, _msg, 0)

        pltpu.sync_copy(hs_v, agg_s.at[idx_v.at[3]], add=True)
        return carry

    nk = jnp.where(wid < NCHUNKS_ALL - NTILES * (NCHUNKS_ALL // NTILES),
                   NCHUNKS_ALL // NTILES + 1, NCHUNKS_ALL // NTILES)
    lax.fori_loop(0, nk, _chunk, 0)
    plsc.subcore_barrier()

    @pl.when(sid < NS - 1)
    def _():
        pltpu.sync_copy(agg_s.at[pl.ds(sid * RPT, RPT)],
                        out_h.at[cid, pl.ds(sid * RPT, RPT)])

    @pl.when(sid == NS - 1)
    def _():
        pltpu.sync_copy(agg_s.at[pl.ds(sid * RPT, RPT_LAST)],
                        out_h.at[cid, pl.ds(sid * RPT, RPT_LAST)])


def _sc_edges(idxpack, q_rel, hidden, re_, wv, planes):
    mesh = plsc.VectorSubcoreMesh(core_axis_name="c", subcore_axis_name="s")
    f = pl.kernel(
        _sc_body,
        out_type=jax.ShapeDtypeStruct((NC, N, D), jnp.float32),
        mesh=mesh,
        compiler_params=pltpu.CompilerParams(needs_layout_passes=False),
        scratch_types=[
            pltpu.VMEM((4, CH), jnp.int32),
            pltpu.VMEM((CH,), jnp.int32),
            pltpu.VMEM((15, CH), jnp.float32),
            pltpu.VMEM((CH, D), jnp.float32),
            pltpu.VMEM((CH, D), jnp.float32),
            pltpu.VMEM((CH + 16,), jnp.float32),
            pltpu.VMEM((16,), jnp.float32),
            pltpu.VMEM_SHARED((N, D), jnp.float32),
        ],
    )
    return f(idxpack, q_rel, hidden, re_, wv, *planes)


def _final_body(p0, p1, wh, delta, out_o):
    acc = p0[...] + p1[...] + delta[0, 0]
    out_o[...] = jnp.dot(acc, wh[...], preferred_element_type=jnp.float32)


def _final(p0, p1, wh, delta):
    bs = 1000
    return pl.pallas_call(
        _final_body,
        grid=(N // bs,),
        in_specs=[
            pl.BlockSpec((bs, D), lambda i: (i, 0)),
            pl.BlockSpec((bs, D), lambda i: (i, 0)),
            pl.BlockSpec((D, D), lambda i: (0, 0)),
            pl.BlockSpec(memory_space=pltpu.SMEM),
        ],
        out_specs=pl.BlockSpec((bs, D), lambda i: (i, 0)),
        out_shape=jax.ShapeDtypeStruct((N, D), jnp.float32),
    )(p0, p1, wh, delta)


def kernel(q_sub, q_rel, hidden, edges, n_node, old_nodes_new_idx, rela_embed,
           Ws_attn, Wr_attn, Wqr_attn_W, Wqr_attn_b, w_alpha_W, w_alpha_b,
           W_h):
    idxpack = (edges[:, jnp.array([4, 2, 0, 5])]
               .reshape(NCHUNKS_ALL, CH, 4)
               .transpose(0, 2, 1))
    re_ = rela_embed[:N]

    ws_p = jnp.pad(Ws_attn, ((0, 0), (0, PADW - 5)))
    wr_p = jnp.pad(Wr_attn, ((0, 0), (0, PADW - 5)))
    wq_p = jnp.pad(Wqr_attn_W, ((0, 0), (0, PADW - 5)))
    bq_p = jnp.pad(Wqr_attn_b, (0, PADW - 5)).reshape(1, PADW)

    ps, pr, pq = _proj_tables(hidden, re_, ws_p, wr_p, wq_p, bq_p)
    planes = tuple(ps[:, j] for j in range(5)) \
        + tuple(pr[:, j] for j in range(5)) \
        + tuple(pq[:, j] for j in range(5))

    wv = jnp.concatenate([w_alpha_W[:, 0], w_alpha_b,
                          jnp.zeros((10,), jnp.float32)])

    partials = _sc_edges(idxpack, q_rel.astype(jnp.int32),
                         hidden, re_, wv, planes)

    delta = jnp.asarray(n_node - N, jnp.float32).reshape(1, 1)
    return _final(partials[0], partials[1], W_h, delta)


# speedup vs baseline: 5.5279x; 2.2011x over previous
import functools

import jax
import jax.numpy as jnp
from jax import lax
from jax.experimental import pallas as pl
from jax.experimental.pallas import tpu as pltpu
from jax.experimental.pallas import tpu_sc as plsc

N = 10000
E = 320000
D = 128
PADW = 16
NC, NS = 2, 16
NTILES = NC * NS
CH = 128
NCHUNKS_ALL = E // CH
RPT = 632
RPT_LAST = N - (NS - 1) * RPT


def _proj_body(hid, re_, ws, wr, wq, bq, ps_o, pr_o, pq_o):
    ps_o[...] = jnp.dot(hid[...], ws[...], preferred_element_type=jnp.float32)
    pr_o[...] = jnp.dot(re_[...], wr[...], preferred_element_type=jnp.float32)
    pq_o[...] = (jnp.dot(re_[...], wq[...], preferred_element_type=jnp.float32)
                 + bq[...])


def _proj_tables(hidden, re_, ws_p, wr_p, wq_p, bq_p):
    bs = 1000
    grid = (N // bs,)
    return pl.pallas_call(
        _proj_body,
        grid=grid,
        in_specs=[
            pl.BlockSpec((bs, D), lambda i: (i, 0)),
            pl.BlockSpec((bs, D), lambda i: (i, 0)),
            pl.BlockSpec((D, PADW), lambda i: (0, 0)),
            pl.BlockSpec((D, PADW), lambda i: (0, 0)),
            pl.BlockSpec((D, PADW), lambda i: (0, 0)),
            pl.BlockSpec((1, PADW), lambda i: (0, 0)),
        ],
        out_specs=[
            pl.BlockSpec((bs, PADW), lambda i: (i, 0)),
            pl.BlockSpec((bs, PADW), lambda i: (i, 0)),
            pl.BlockSpec((bs, PADW), lambda i: (i, 0)),
        ],
        out_shape=[jax.ShapeDtypeStruct((N, PADW), jnp.float32)] * 3,
    )(hidden, re_, ws_p, wr_p, wq_p, bq_p)


def _sc_body(idx_h, qrel_h, hid_h, re_h, wv_h,
             p0_h, p1_h, p2_h, p3_h, p4_h, p5_h, p6_h, p7_h, p8_h, p9_h,
             p10_h, p11_h, p12_h, p13_h, p14_h, out_h,
             idx_v, qi_v, att_v, hs_v, hr_v, alpha_v, wv_v, agg_s,
             sem_a, sem_q):
    planes = (p0_h, p1_h, p2_h, p3_h, p4_h, p5_h, p6_h, p7_h, p8_h, p9_h,
              p10_h, p11_h, p12_h, p13_h, p14_h)
    cid = lax.axis_index("c")
    sid = lax.axis_index("s")
    wid = cid * NS + sid

    zf = jnp.zeros((16,), jnp.float32)

    pltpu.sync_copy(wv_h, wv_v)

    def _zrow(rr, carry):
        for dblk in range(D // 16):
            hs_v[rr, pl.ds(dblk * 16, 16)] = zf
        return carry
    lax.fori_loop(0, CH, _zrow, 0)

    def _zero_rows(start, count):
        nfull = count // CH
        def _zagg(k, carry):
            pltpu.sync_copy(hs_v, agg_s.at[pl.ds(start + k * CH, CH)])
            return carry
        lax.fori_loop(0, nfull, _zagg, 0)
        rem = count - nfull * CH
        if rem:
            pltpu.sync_copy(hs_v.at[pl.ds(0, rem)],
                            agg_s.at[pl.ds(start + nfull * CH, rem)])

    @pl.when(sid < NS - 1)
    def _():
        _zero_rows(sid * RPT, RPT)

    @pl.when(sid == NS - 1)
    def _():
        _zero_rows(sid * RPT, RPT_LAST)
    plsc.subcore_barrier()

    def _chunk(k, carry):
        c = wid + NTILES * k
        pltpu.sync_copy(idx_h.at[c], idx_v)
        sub_s = idx_v.at[0]
        rel_s = idx_v.at[1]
        hq = pltpu.async_copy(qrel_h.at[idx_v.at[2]], qi_v, sem_q)
        hs = []
        for j in range(5):
            hs.append(pltpu.async_copy(planes[j].at[sub_s], att_v.at[j],
                                       sem_a))
            hs.append(pltpu.async_copy(planes[5 + j].at[rel_s],
                                       att_v.at[5 + j], sem_a))
        hs.append(pltpu.async_copy(hid_h.at[sub_s], hs_v, sem_a))
        hs.append(pltpu.async_copy(re_h.at[rel_s], hr_v, sem_a))
        hq.wait()
        for j in range(5):
            hs.append(pltpu.async_copy(planes[10 + j].at[qi_v],
                                       att_v.at[10 + j], sem_a))
        for h in hs:
            h.wait()

        wvec = wv_v[...]
        for g in range(CH // 16):
            logit = jnp.broadcast_to(wvec[5], (16,))
            for j in range(5):
                sj = att_v[j, pl.ds(g * 16, 16)]
                rj = att_v[5 + j, pl.ds(g * 16, 16)]
                qj = att_v[10 + j, pl.ds(g * 16, 16)]
                logit = logit + jnp.maximum(sj + rj + qj, 0.0) * wvec[j]
            alpha = 1.0 / (1.0 + jnp.exp(-logit))
            alpha_v[pl.ds(g * 16, 16)] = alpha

        def _msg(e, c2):
            a = jnp.broadcast_to(alpha_v[pl.ds(e, 16)][0], (16,))
            for dblk in range(D // 16):
                h1 = hs_v[e, pl.ds(dblk * 16, 16)]
                h2 = hr_v[e, pl.ds(dblk * 16, 16)]
                hs_v[e, pl.ds(dblk * 16, 16)] = a * (h1 + h2)
            return c2
        lax.fori_loop(0, CH, _msg, 0)

        pltpu.sync_copy(hs_v, agg_s.at[idx_v.at[3]], add=True)
        return carry

    nk = jnp.where(wid < NCHUNKS_ALL - NTILES * (NCHUNKS_ALL // NTILES),
                   NCHUNKS_ALL // NTILES + 1, NCHUNKS_ALL // NTILES)
    lax.fori_loop(0, nk, _chunk, 0)
    plsc.subcore_barrier()

    @pl.when(sid < NS - 1)
    def _():
        pltpu.sync_copy(agg_s.at[pl.ds(sid * RPT, RPT)],
                        out_h.at[cid, pl.ds(sid * RPT, RPT)])

    @pl.when(sid == NS - 1)
    def _():
        pltpu.sync_copy(agg_s.at[pl.ds(sid * RPT, RPT_LAST)],
                        out_h.at[cid, pl.ds(sid * RPT, RPT_LAST)])


def _sc_edges(idxpack, q_rel, hidden, re_, wv, planes):
    mesh = plsc.VectorSubcoreMesh(core_axis_name="c", subcore_axis_name="s")
    f = pl.kernel(
        _sc_body,
        out_type=jax.ShapeDtypeStruct((NC, N, D), jnp.float32),
        mesh=mesh,
        compiler_params=pltpu.CompilerParams(needs_layout_passes=False),
        scratch_types=[
            pltpu.VMEM((4, CH), jnp.int32),
            pltpu.VMEM((CH,), jnp.int32),
            pltpu.VMEM((15, CH), jnp.float32),
            pltpu.VMEM((CH, D), jnp.float32),
            pltpu.VMEM((CH, D), jnp.float32),
            pltpu.VMEM((CH + 16,), jnp.float32),
            pltpu.VMEM((16,), jnp.float32),
            pltpu.VMEM_SHARED((N, D), jnp.float32),
            pltpu.SemaphoreType.DMA,
            pltpu.SemaphoreType.DMA,
        ],
    )
    return f(idxpack, q_rel, hidden, re_, wv, *planes)


def _final_body(p0, p1, wh, delta, out_o):
    acc = p0[...] + p1[...] + delta[0, 0]
    out_o[...] = jnp.dot(acc, wh[...], preferred_element_type=jnp.float32)


def _final(p0, p1, wh, delta):
    bs = 1000
    return pl.pallas_call(
        _final_body,
        grid=(N // bs,),
        in_specs=[
            pl.BlockSpec((bs, D), lambda i: (i, 0)),
            pl.BlockSpec((bs, D), lambda i: (i, 0)),
            pl.BlockSpec((D, D), lambda i: (0, 0)),
            pl.BlockSpec(memory_space=pltpu.SMEM),
        ],
        out_specs=pl.BlockSpec((bs, D), lambda i: (i, 0)),
        out_shape=jax.ShapeDtypeStruct((N, D), jnp.float32),
    )(p0, p1, wh, delta)


def kernel(q_sub, q_rel, hidden, edges, n_node, old_nodes_new_idx, rela_embed,
           Ws_attn, Wr_attn, Wqr_attn_W, Wqr_attn_b, w_alpha_W, w_alpha_b,
           W_h):
    idxpack = (edges[:, jnp.array([4, 2, 0, 5])]
               .reshape(NCHUNKS_ALL, CH, 4)
               .transpose(0, 2, 1))
    re_ = rela_embed[:N]

    ws_p = jnp.pad(Ws_attn, ((0, 0), (0, PADW - 5)))
    wr_p = jnp.pad(Wr_attn, ((0, 0), (0, PADW - 5)))
    wq_p = jnp.pad(Wqr_attn_W, ((0, 0), (0, PADW - 5)))
    bq_p = jnp.pad(Wqr_attn_b, (0, PADW - 5)).reshape(1, PADW)

    ps, pr, pq = _proj_tables(hidden, re_, ws_p, wr_p, wq_p, bq_p)
    planes = tuple(ps[:, j] for j in range(5)) \
        + tuple(pr[:, j] for j in range(5)) \
        + tuple(pq[:, j] for j in range(5))

    wv = jnp.concatenate([w_alpha_W[:, 0], w_alpha_b,
                          jnp.zeros((10,), jnp.float32)])

    partials = _sc_edges(idxpack, q_rel.astype(jnp.int32),
                         hidden, re_, wv, planes)

    delta = jnp.asarray(n_node - N, jnp.float32).reshape(1, 1)
    return _final(partials[0], partials[1], W_h, delta)


# speedup vs baseline: 7.6557x; 1.3849x over previous
import functools

import jax
import jax.numpy as jnp
from jax import lax
from jax.experimental import pallas as pl
from jax.experimental.pallas import tpu as pltpu
from jax.experimental.pallas import tpu_sc as plsc

N = 10000
E = 320000
D = 128
PADW = 16
NC, NS = 2, 16
NTILES = NC * NS
CH = 80
NCHUNKS_ALL = E // CH
KPT = NCHUNKS_ALL // NTILES
RPT = 632
RPT_LAST = N - (NS - 1) * RPT


def _proj_body(hid, re_, ws, wr, wq, bq, ps_o, pr_o, pq_o):
    ps_o[...] = jnp.dot(hid[...], ws[...], preferred_element_type=jnp.float32)
    pr_o[...] = jnp.dot(re_[...], wr[...], preferred_element_type=jnp.float32)
    pq_o[...] = (jnp.dot(re_[...], wq[...], preferred_element_type=jnp.float32)
                 + bq[...])


def _proj_tables(hidden, re_, ws_p, wr_p, wq_p, bq_p):
    bs = 1000
    grid = (N // bs,)
    return pl.pallas_call(
        _proj_body,
        grid=grid,
        in_specs=[
            pl.BlockSpec((bs, D), lambda i: (i, 0)),
            pl.BlockSpec((bs, D), lambda i: (i, 0)),
            pl.BlockSpec((D, PADW), lambda i: (0, 0)),
            pl.BlockSpec((D, PADW), lambda i: (0, 0)),
            pl.BlockSpec((D, PADW), lambda i: (0, 0)),
            pl.BlockSpec((1, PADW), lambda i: (0, 0)),
        ],
        out_specs=[
            pl.BlockSpec((bs, PADW), lambda i: (i, 0)),
            pl.BlockSpec((bs, PADW), lambda i: (i, 0)),
            pl.BlockSpec((bs, PADW), lambda i: (i, 0)),
        ],
        out_shape=[jax.ShapeDtypeStruct((N, PADW), jnp.float32)] * 3,
    )(hidden, re_, ws_p, wr_p, wq_p, bq_p)


def _sc_body(idx_h, qrel_h, hid_h, re_h, wv_h,
             p0_h, p1_h, p2_h, p3_h, p4_h, p5_h, p6_h, p7_h, p8_h, p9_h,
             p10_h, p11_h, p12_h, p13_h, p14_h, out_h,
             idx_v0, idx_v1, qi_v0, qi_v1, att_v0, att_v1,
             hs_v0, hs_v1, hr_v0, hr_v1, alpha_v, wv_v, agg_s,
             sem_a0, sem_a1, sem_q0, sem_q1):
    planes = (p0_h, p1_h, p2_h, p3_h, p4_h, p5_h, p6_h, p7_h, p8_h, p9_h,
              p10_h, p11_h, p12_h, p13_h, p14_h)
    cid = lax.axis_index("c")
    sid = lax.axis_index("s")
    wid = cid * NS + sid

    sets = (
        dict(idx=idx_v0, qi=qi_v0, att=att_v0, hs=hs_v0, hr=hr_v0,
             sem_a=sem_a0, sem_q=sem_q0),
        dict(idx=idx_v1, qi=qi_v1, att=att_v1, hs=hs_v1, hr=hr_v1,
             sem_a=sem_a1, sem_q=sem_q1),
    )

    zf = jnp.zeros((16,), jnp.float32)

    pltpu.sync_copy(wv_h, wv_v)

    hs_v = hs_v0
    def _zrow(rr, carry):
        for dblk in range(D // 16):
            hs_v[rr, pl.ds(dblk * 16, 16)] = zf
        return carry
    lax.fori_loop(0, CH, _zrow, 0)

    def _zero_rows(start, count):
        nfull = count // CH
        def _zagg(k, carry):
            pltpu.sync_copy(hs_v, agg_s.at[pl.ds(start + k * CH, CH)])
            return carry
        lax.fori_loop(0, nfull, _zagg, 0)
        rem = count - nfull * CH
        if rem:
            pltpu.sync_copy(hs_v.at[pl.ds(0, rem)],
                            agg_s.at[pl.ds(start + nfull * CH, rem)])

    @pl.when(sid < NS - 1)
    def _():
        _zero_rows(sid * RPT, RPT)

    @pl.when(sid == NS - 1)
    def _():
        _zero_rows(sid * RPT, RPT_LAST)
    plsc.subcore_barrier()

    def _main_copies(st):
        cps = []
        for j in range(5):
            cps.append((planes[j].at[st["idx"].at[0]], st["att"].at[j],
                        st["sem_a"]))
            cps.append((planes[5 + j].at[st["idx"].at[1]],
                        st["att"].at[5 + j], st["sem_a"]))
        cps.append((hid_h.at[st["idx"].at[0]], st["hs"], st["sem_a"]))
        cps.append((re_h.at[st["idx"].at[1]], st["hr"], st["sem_a"]))
        return cps

    def _fire1(k, st):
        c = wid + NTILES * k
        pltpu.sync_copy(idx_h.at[c], st["idx"])
        pltpu.async_copy(qrel_h.at[st["idx"].at[2]], st["qi"], st["sem_q"])
        for t in _main_copies(st):
            pltpu.async_copy(*t)

    def _fire2(st):
        pltpu.make_async_copy(qrel_h.at[st["idx"].at[2]], st["qi"],
                              st["sem_q"]).wait()
        for j in range(5):
            pltpu.async_copy(planes[10 + j].at[st["qi"]],
                             st["att"].at[10 + j], st["sem_a"])

    def _consume(st):
        for t in _main_copies(st):
            pltpu.make_async_copy(*t).wait()
        for j in range(5):
            pltpu.make_async_copy(planes[10 + j].at[st["qi"]],
                                  st["att"].at[10 + j], st["sem_a"]).wait()
        att = st["att"]
        wvec = wv_v[...]
        for g in range(CH // 16):
            logit = jnp.broadcast_to(wvec[5], (16,))
            for j in range(5):
                sj = att[j, pl.ds(g * 16, 16)]
                rj = att[5 + j, pl.ds(g * 16, 16)]
                qj = att[10 + j, pl.ds(g * 16, 16)]
                logit = logit + jnp.maximum(sj + rj + qj, 0.0) * wvec[j]
            alpha = 1.0 / (1.0 + jnp.exp(-logit))
            alpha_v[pl.ds(g * 16, 16)] = alpha

        hsb, hrb = st["hs"], st["hr"]
        def _msg(e, c2):
            a = jnp.broadcast_to(alpha_v[pl.ds(e, 16)][0], (16,))
            for dblk in range(D // 16):
                h1 = hsb[e, pl.ds(dblk * 16, 16)]
                h2 = hrb[e, pl.ds(dblk * 16, 16)]
                hsb[e, pl.ds(dblk * 16, 16)] = a * (h1 + h2)
            return c2
        lax.fori_loop(0, CH, _msg, 0)

        pltpu.sync_copy(hsb, agg_s.at[st["idx"].at[3]], add=True)

    _fire1(0, sets[0])

    def _pair(i, carry):
        k0 = 2 * i
        _fire2(sets[0])
        _fire1(k0 + 1, sets[1])
        _consume(sets[0])
        _fire2(sets[1])
        _fire1(k0 + 2, sets[0])
        _consume(sets[1])
        return carry

    lax.fori_loop(0, (KPT - 1) // 2, _pair, 0)
    _fire2(sets[0])
    _consume(sets[0])
    plsc.subcore_barrier()

    @pl.when(sid < NS - 1)
    def _():
        pltpu.sync_copy(agg_s.at[pl.ds(sid * RPT, RPT)],
                        out_h.at[cid, pl.ds(sid * RPT, RPT)])

    @pl.when(sid == NS - 1)
    def _():
        pltpu.sync_copy(agg_s.at[pl.ds(sid * RPT, RPT_LAST)],
                        out_h.at[cid, pl.ds(sid * RPT, RPT_LAST)])


def _sc_edges(idxpack, q_rel, hidden, re_, wv, planes):
    mesh = plsc.VectorSubcoreMesh(core_axis_name="c", subcore_axis_name="s")
    f = pl.kernel(
        _sc_body,
        out_type=jax.ShapeDtypeStruct((NC, N, D), jnp.float32),
        mesh=mesh,
        compiler_params=pltpu.CompilerParams(needs_layout_passes=False),
        scratch_types=[
            pltpu.VMEM((4, CH), jnp.int32),
            pltpu.VMEM((4, CH), jnp.int32),
            pltpu.VMEM((CH,), jnp.int32),
            pltpu.VMEM((CH,), jnp.int32),
            pltpu.VMEM((15, CH), jnp.float32),
            pltpu.VMEM((15, CH), jnp.float32),
            pltpu.VMEM((CH, D), jnp.float32),
            pltpu.VMEM((CH, D), jnp.float32),
            pltpu.VMEM((CH, D), jnp.float32),
            pltpu.VMEM((CH, D), jnp.float32),
            pltpu.VMEM((CH + 16,), jnp.float32),
            pltpu.VMEM((16,), jnp.float32),
            pltpu.VMEM_SHARED((N, D), jnp.float32),
            pltpu.SemaphoreType.DMA,
            pltpu.SemaphoreType.DMA,
            pltpu.SemaphoreType.DMA,
            pltpu.SemaphoreType.DMA,
        ],
    )
    return f(idxpack, q_rel, hidden, re_, wv, *planes)


def _final_body(p0, p1, wh, delta, out_o):
    acc = p0[...] + p1[...] + delta[0, 0]
    out_o[...] = jnp.dot(acc, wh[...], preferred_element_type=jnp.float32)


def _final(p0, p1, wh, delta):
    bs = 1000
    return pl.pallas_call(
        _final_body,
        grid=(N // bs,),
        in_specs=[
            pl.BlockSpec((bs, D), lambda i: (i, 0)),
            pl.BlockSpec((bs, D), lambda i: (i, 0)),
            pl.BlockSpec((D, D), lambda i: (0, 0)),
            pl.BlockSpec(memory_space=pltpu.SMEM),
        ],
        out_specs=pl.BlockSpec((bs, D), lambda i: (i, 0)),
        out_shape=jax.ShapeDtypeStruct((N, D), jnp.float32),
    )(p0, p1, wh, delta)


def kernel(q_sub, q_rel, hidden, edges, n_node, old_nodes_new_idx, rela_embed,
           Ws_attn, Wr_attn, Wqr_attn_W, Wqr_attn_b, w_alpha_W, w_alpha_b,
           W_h):
    idxpack = (edges[:, jnp.array([4, 2, 0, 5])]
               .reshape(NCHUNKS_ALL, CH, 4)
               .transpose(0, 2, 1))
    re_ = rela_embed[:N]

    ws_p = jnp.pad(Ws_attn, ((0, 0), (0, PADW - 5)))
    wr_p = jnp.pad(Wr_attn, ((0, 0), (0, PADW - 5)))
    wq_p = jnp.pad(Wqr_attn_W, ((0, 0), (0, PADW - 5)))
    bq_p = jnp.pad(Wqr_attn_b, (0, PADW - 5)).reshape(1, PADW)

    ps, pr, pq = _proj_tables(hidden, re_, ws_p, wr_p, wq_p, bq_p)
    planes = tuple(ps[:, j] for j in range(5)) \
        + tuple(pr[:, j] for j in range(5)) \
        + tuple(pq[:, j] for j in range(5))

    wv = jnp.concatenate([w_alpha_W[:, 0], w_alpha_b,
                          jnp.zeros((10,), jnp.float32)])

    partials = _sc_edges(idxpack, q_rel.astype(jnp.int32),
                         hidden, re_, wv, planes)

    delta = jnp.asarray(n_node - N, jnp.float32).reshape(1, 1)
    return _final(partials[0], partials[1], W_h, delta)
